# baseline (speedup 1.0000x reference)
import functools

import jax
import jax.numpy as jnp
from jax import lax
from jax.experimental import pallas as pl
from jax.experimental.pallas import tpu as pltpu
from jax.experimental.pallas import tpu_sc as plsc

DIM = 64
LANES = 16
NW = 32
B = 16384
B_PER_W = B // NW
CH = 128
NCH = B_PER_W // CH
GROUPS = B_PER_W // LANES


def _make_kernel():
    @functools.partial(
        pl.kernel,
        mesh=plsc.VectorSubcoreMesh(core_axis_name="c", subcore_axis_name="s"),
        out_type=jax.ShapeDtypeStruct((B,), jnp.float32),
        compiler_params=pltpu.CompilerParams(use_tc_tiling_on_sc=False),
        scratch_types=[
            pltpu.VMEM((NCH, CH), jnp.int32),
            pltpu.VMEM((NCH, CH), jnp.int32),
            pltpu.VMEM((NCH, CH), jnp.int32),
            pltpu.VMEM((B_PER_W, DIM), jnp.float32),
            pltpu.VMEM((B_PER_W, DIM), jnp.float32),
            pltpu.VMEM((B_PER_W, DIM), jnp.float32),
            pltpu.VMEM((B_PER_W,), jnp.float32),
            pltpu.SemaphoreType.DMA,
        ],
    )
    def distmult(h_idx_hbm, t_idx_hbm, r_idx_hbm, ent_hbm, rel_hbm, out_hbm,
                 hidx_v, tidx_v, ridx_v, h_v, t_v, r_v, out_v, sem):
        wid = lax.axis_index("s") * 2 + lax.axis_index("c")
        row0 = wid * NCH
        pltpu.sync_copy(h_idx_hbm.at[pl.ds(row0, NCH)], hidx_v)
        pltpu.sync_copy(t_idx_hbm.at[pl.ds(row0, NCH)], tidx_v)
        pltpu.sync_copy(r_idx_hbm.at[pl.ds(row0, NCH)], ridx_v)

        copies = []
        for c in range(NCH):
            dst = pl.ds(c * CH, CH)
            copies.append(pltpu.async_copy(ent_hbm.at[hidx_v.at[c]], h_v.at[dst], sem))
            copies.append(pltpu.async_copy(ent_hbm.at[tidx_v.at[c]], t_v.at[dst], sem))
            copies.append(pltpu.async_copy(rel_hbm.at[ridx_v.at[c]], r_v.at[dst], sem))
        for cp in copies:
            cp.wait()

        lane = jnp.arange(LANES, dtype=jnp.int32)
        perms = [(lane ^ (1 << b)).reshape(LANES, 1) for b in range(4)]
        dnums = lax.GatherDimensionNumbers(
            offset_dims=(), collapsed_slice_dims=(0,), start_index_map=(0,))

        def shuffle(x, pm):
            return lax.gather(x, pm, dnums, (1,),
                              mode=lax.GatherScatterMode.PROMISE_IN_BOUNDS)

        def group(g, _):
            acc = jnp.zeros((LANES,), jnp.float32)
            for rr in range(LANES):
                r_i = g * LANES + rr
                acc4 = None
                for k in range(DIM // LANES):
                    sl = pl.ds(k * LANES, LANES)
                    p = h_v[r_i, sl] * t_v[r_i, sl] * r_v[r_i, sl]
                    acc4 = p if acc4 is None else acc4 + p
                for pm in perms:
                    acc4 = acc4 + shuffle(acc4, pm)
                acc = jnp.where(lane == rr, acc4, acc)
            out_v[pl.ds(g * LANES, LANES)] = acc
            return 0

        lax.fori_loop(0, GROUPS, group, 0)
        pltpu.sync_copy(out_v, out_hbm.at[pl.ds(wid * B_PER_W, B_PER_W)])

    return distmult


_distmult = _make_kernel()


def kernel(batch_h, batch_t, batch_r, ent_embeddings, rel_embeddings):
    h2 = batch_h.astype(jnp.int32).reshape(NW * NCH, CH)
    t2 = batch_t.astype(jnp.int32).reshape(NW * NCH, CH)
    r2 = batch_r.astype(jnp.int32).reshape(NW * NCH, CH)
    return _distmult(h2, t2, r2, ent_embeddings, rel_embeddings)
